# baseline (speedup 1.0000x reference)
import functools

import jax
import jax.numpy as jnp
from jax import lax
from jax.experimental import pallas as pl
from jax.experimental.pallas import tpu as pltpu
from jax.experimental.pallas import tpu_sc as plsc

H = 1024
I = 2048
E = 7
T = 4096
BLK = 256
NB = 39
NSLOTS = NB * BLK
NC, NS, L = 2, 16, 16
NW = NC * NS
TPW = T // NW


def _router_body(x_ref, rw_ref, bias_ref, e1_ref, e2_ref, w1_ref, w2_ref,
                 cnt_ref):
    xb = x_ref[...]
    logits = jnp.dot(xb, rw_ref[...], preferred_element_type=jnp.float32)
    probs = jax.nn.sigmoid(logits + bias_ref[...])
    lanes = lax.broadcasted_iota(jnp.int32, probs.shape, 1)
    m1 = jnp.max(probs, axis=1, keepdims=True)
    i1 = jnp.min(jnp.where(probs >= m1, lanes, 127), axis=1, keepdims=True)
    probs2 = jnp.where(lanes == i1, -1.0, probs)
    m2 = jnp.max(probs2, axis=1, keepdims=True)
    i2 = jnp.min(jnp.where(probs2 >= m2, lanes, 127), axis=1, keepdims=True)
    s = m1 + m2
    e1_ref[...] = i1
    e2_ref[...] = i2
    w1_ref[...] = m1 / s
    w2_ref[...] = m2 / s
    lanes16 = lax.broadcasted_iota(jnp.int32, (512, L), 1)
    onehot = ((lanes16 == i1) | (lanes16 == i2)).astype(jnp.float32)
    seg = lax.broadcasted_iota(jnp.int32, (4, 512), 0)
    tok = lax.broadcasted_iota(jnp.int32, (4, 512), 1)
    sel = (tok // TPW == seg).astype(jnp.float32)
    cnt = jnp.dot(sel, onehot, preferred_element_type=jnp.float32)
    cnt_ref[...] = cnt[None].astype(jnp.int32)


def _router(x2d, rw, bias2d):
    tb = 512
    grid = (T // tb,)
    return pl.pallas_call(
        _router_body,
        grid=grid,
        in_specs=[
            pl.BlockSpec((tb, H), lambda b: (b, 0)),
            pl.BlockSpec((H, E), lambda b: (0, 0)),
            pl.BlockSpec((1, E), lambda b: (0, 0)),
        ],
        out_specs=[
            pl.BlockSpec((tb, 1), lambda b: (b, 0)),
            pl.BlockSpec((tb, 1), lambda b: (b, 0)),
            pl.BlockSpec((tb, 1), lambda b: (b, 0)),
            pl.BlockSpec((tb, 1), lambda b: (b, 0)),
            pl.BlockSpec((1, 4, L), lambda b: (b, 0, 0)),
        ],
        out_shape=[
            jax.ShapeDtypeStruct((T, 1), jnp.int32),
            jax.ShapeDtypeStruct((T, 1), jnp.int32),
            jax.ShapeDtypeStruct((T, 1), jnp.float32),
            jax.ShapeDtypeStruct((T, 1), jnp.float32),
            jax.ShapeDtypeStruct((T // tb, 4, L), jnp.int32),
        ],
    )(x2d, rw, bias2d)


def _dispatch_body(x_hbm, e1_hbm, e2_hbm, cnt_hbm,
                   disp_hbm, pos1_hbm, pos2_hbm, bex_hbm, nb_hbm,
                   cnt_v, e1_v, e2_v, rows_v, rows2_v, bex_v, nb_v,
                   p1c0, p1c1, p1c2, p1c3, p2c0, p2c1, p2c2, p2c3,
                   sem, semla, semlb):
    cid = lax.axis_index("c")
    sid = lax.axis_index("s")
    wid = sid * NC + cid
    lane = lax.iota(jnp.int32, L)

    pltpu.sync_copy(cnt_hbm, cnt_v)
    totals = cnt_v[0, :]
    for w in range(1, NW):
        totals = totals + cnt_v[w, :]
    pad = ((totals + (BLK - 1)) >> 8) << 8
    cum_incl = plsc.cumsum(pad)
    start = cum_incl - pad
    base = start
    for w in range(NW):
        base = base + jnp.where(w < wid, cnt_v[w, :], 0)

    t0 = wid * TPW
    pltpu.sync_copy(e1_hbm.at[pl.ds(t0, TPW)], e1_v)
    pltpu.sync_copy(e2_hbm.at[pl.ds(t0, TPW)], e2_v)

    p1c = (p1c0, p1c1, p1c2, p1c3)
    p2c = (p2c0, p2c1, p2c2, p2c3)
    for stream, chunks in ((e1_v, p1c), (e2_v, p2c)):
        for v in range(TPW // L):
            ev = stream[pl.ds(v * L, L)]
            pos = jnp.zeros((L,), jnp.int32)
            for e in range(E):
                mask = ev == e
                incl = plsc.cumsum(jnp.where(mask, 1, 0))
                base_e = jnp.sum(jnp.where(lane == e, base, 0))
                pos = jnp.where(mask, base_e + incl - 1, pos)
                base = base + jnp.where(lane == e, jnp.max(incl), 0)
            chunks[v // 2][pl.ds((v % 2) * L, L)] = pos

    rows = (rows_v, rows2_v)
    semls = (semla, semlb)
    loads = {}
    scats = {}

    def fire_load(c):
        loads[c] = pltpu.async_copy(
            x_hbm.at[pl.ds(t0 + c * 32, 32)], rows[c % 2], semls[c % 2])

    fire_load(0)
    for c in range(4):
        if c < 4 - 1:
            if c - 1 >= 0:
                for dsc in scats.pop(c - 1):
                    dsc.wait()
            fire_load(c + 1)
        loads.pop(c).wait()
        scats[c] = (
            pltpu.async_copy(rows[c % 2], disp_hbm.at[p1c[c]], sem),
            pltpu.async_copy(rows[c % 2], disp_hbm.at[p2c[c]], sem),
        )
        tc = t0 + c * 32
        pltpu.sync_copy(p1c[c], pos1_hbm.at[pl.ds(tc, 32)])
        pltpu.sync_copy(p2c[c], pos2_hbm.at[pl.ds(tc, 32)])
    for c in (2, 3):
        for dsc in scats.pop(c):
            dsc.wait()

    @pl.when(wid == 0)
    def _():
        total_pad = jnp.max(jnp.where(lane < E, cum_incl, 0))
        for j in range(3):
            bidx = (lane + j * L) * BLK
            acc = jnp.zeros((L,), jnp.int32)
            for e in range(E):
                ci = jnp.sum(jnp.where(lane == e, cum_incl, 0))
                acc = acc + jnp.where(bidx >= ci, 1, 0)
            bex_v[pl.ds(j * L, L)] = jnp.minimum(acc, E - 1)
        nb_v[...] = jnp.where(lane == 0, total_pad >> 8, 0)
        pltpu.sync_copy(bex_v, bex_hbm)
        pltpu.sync_copy(nb_v, nb_hbm)


def _dispatch(x2d, e1, e2, cnt):
    mesh = plsc.VectorSubcoreMesh(core_axis_name="c", subcore_axis_name="s")
    f = pl.kernel(
        _dispatch_body,
        out_type=[
            jax.ShapeDtypeStruct((NSLOTS, H), jnp.float32),
            jax.ShapeDtypeStruct((T,), jnp.int32),
            jax.ShapeDtypeStruct((T,), jnp.int32),
            jax.ShapeDtypeStruct((48,), jnp.int32),
            jax.ShapeDtypeStruct((L,), jnp.int32),
        ],
        mesh=mesh,
        scratch_types=[
            pltpu.VMEM((NW, L), jnp.int32),
            pltpu.VMEM((TPW,), jnp.int32),
            pltpu.VMEM((TPW,), jnp.int32),
            pltpu.VMEM((32, H), jnp.float32),
            pltpu.VMEM((32, H), jnp.float32),
            pltpu.VMEM((48,), jnp.int32),
            pltpu.VMEM((L,), jnp.int32),
        ] + [pltpu.VMEM((32,), jnp.int32) for _ in range(8)]
        + [pltpu.SemaphoreType.DMA, pltpu.SemaphoreType.DMA,
           pltpu.SemaphoreType.DMA],
        compiler_params=pltpu.CompilerParams(needs_layout_passes=False),
    )
    return f(x2d, e1, e2, cnt)


def _ffn_body(x_ref, g_ref, u_ref, d_ref, y_ref):
    xb = x_ref[...].astype(jnp.bfloat16)
    a = jnp.dot(xb, g_ref[...], preferred_element_type=jnp.float32)
    b = jnp.dot(xb, u_ref[...], preferred_element_type=jnp.float32)
    inter = (jax.nn.silu(a) * b).astype(jnp.bfloat16)
    y_ref[...] = jnp.dot(inter, d_ref[...], preferred_element_type=jnp.float32)


def _shared_ffn(x2d, gw, uw, dw):
    grid = (T // BLK,)
    return pl.pallas_call(
        _ffn_body,
        grid=grid,
        in_specs=[
            pl.BlockSpec((BLK, H), lambda b: (b, 0)),
            pl.BlockSpec((H, I), lambda b: (0, 0)),
            pl.BlockSpec((H, I), lambda b: (0, 0)),
            pl.BlockSpec((I, H), lambda b: (0, 0)),
        ],
        out_specs=pl.BlockSpec((BLK, H), lambda b: (b, 0)),
        out_shape=jax.ShapeDtypeStruct((T, H), jnp.float32),
    )(x2d, gw, uw, dw)


def _routed_body(bex_ref, nb_ref, x_ref, g_ref, u_ref, d_ref, y_ref):
    b = pl.program_id(0)

    @pl.when(b < nb_ref[0])
    def _():
        xb = x_ref[...].astype(jnp.bfloat16)
        a = jnp.dot(xb, g_ref[0], preferred_element_type=jnp.float32)
        u = jnp.dot(xb, u_ref[0], preferred_element_type=jnp.float32)
        inter = (jax.nn.silu(a) * u).astype(jnp.bfloat16)
        y_ref[...] = jnp.dot(inter, d_ref[0],
                             preferred_element_type=jnp.float32)


def _routed_ffn(bex, nb, disp, gw, uw, dw):
    grid_spec = pltpu.PrefetchScalarGridSpec(
        num_scalar_prefetch=2,
        grid=(NB,),
        in_specs=[
            pl.BlockSpec((BLK, H), lambda b, bex, nb: (b, 0)),
            pl.BlockSpec((1, H, I), lambda b, bex, nb: (bex[b], 0, 0)),
            pl.BlockSpec((1, H, I), lambda b, bex, nb: (bex[b], 0, 0)),
            pl.BlockSpec((1, I, H), lambda b, bex, nb: (bex[b], 0, 0)),
        ],
        out_specs=pl.BlockSpec((BLK, H), lambda b, bex, nb: (b, 0)),
    )
    return pl.pallas_call(
        _routed_body,
        grid_spec=grid_spec,
        out_shape=jax.ShapeDtypeStruct((NSLOTS, H), jnp.float32),
    )(bex, nb, disp, gw, uw, dw)


def _gather_body(yr_hbm, pos1_hbm, pos2_hbm, g1_hbm, g2_hbm,
                 r1a_v, r1b_v, r2a_v, r2b_v, p1_v, p2_v,
                 p1sa, p1sb, p2sa, p2sb, sema, semb):
    cid = lax.axis_index("c")
    sid = lax.axis_index("s")
    wid = sid * NC + cid
    t0 = wid * TPW

    pltpu.sync_copy(pos1_hbm.at[pl.ds(t0, TPW)], p1_v)
    pltpu.sync_copy(pos2_hbm.at[pl.ds(t0, TPW)], p2_v)

    r1 = (r1a_v, r1b_v)
    r2 = (r2a_v, r2b_v)
    p1s = (p1sa, p1sb)
    p2s = (p2sa, p2sb)
    sems = (sema, semb)
    NCH = TPW // L
    pending = {}

    def fire(c):
        b = c % 2
        p1s[b][...] = p1_v[pl.ds(c * L, L)]
        p2s[b][...] = p2_v[pl.ds(c * L, L)]
        pending[c] = (
            pltpu.async_copy(yr_hbm.at[p1s[b]], r1[b], sems[b]),
            pltpu.async_copy(yr_hbm.at[p2s[b]], r2[b], sems[b]),
        )

    fire(0)
    for c in range(NCH):
        b = c % 2
        if c < NCH - 1:
            fire(c + 1)
        for dsc in pending.pop(c):
            dsc.wait()
        sl = pl.ds(t0 + c * L, L)
        pltpu.sync_copy(r1[b], g1_hbm.at[sl])
        pltpu.sync_copy(r2[b], g2_hbm.at[sl])


def _gather(yr, pos1, pos2):
    mesh = plsc.VectorSubcoreMesh(core_axis_name="c", subcore_axis_name="s")
    f = pl.kernel(
        _gather_body,
        out_type=[
            jax.ShapeDtypeStruct((T, H), jnp.float32),
            jax.ShapeDtypeStruct((T, H), jnp.float32),
        ],
        mesh=mesh,
        scratch_types=[
            pltpu.VMEM((L, H), jnp.float32),
            pltpu.VMEM((L, H), jnp.float32),
            pltpu.VMEM((L, H), jnp.float32),
            pltpu.VMEM((L, H), jnp.float32),
            pltpu.VMEM((TPW,), jnp.int32),
            pltpu.VMEM((TPW,), jnp.int32),
            pltpu.VMEM((L,), jnp.int32),
            pltpu.VMEM((L,), jnp.int32),
            pltpu.VMEM((L,), jnp.int32),
            pltpu.VMEM((L,), jnp.int32),
            pltpu.SemaphoreType.DMA,
            pltpu.SemaphoreType.DMA,
        ],
        compiler_params=pltpu.CompilerParams(needs_layout_passes=False),
    )
    return f(yr, pos1, pos2)


def _combine_body(ys_ref, g1_ref, g2_ref, w1_ref, w2_ref, out_ref):
    out_ref[...] = (ys_ref[...] + w1_ref[...] * g1_ref[...]
                    + w2_ref[...] * g2_ref[...])


def _combine(ys, g1, g2, w1c, w2c):
    grid = (T // BLK,)
    return pl.pallas_call(
        _combine_body,
        grid=grid,
        in_specs=[
            pl.BlockSpec((BLK, H), lambda b: (b, 0)),
            pl.BlockSpec((BLK, H), lambda b: (b, 0)),
            pl.BlockSpec((BLK, H), lambda b: (b, 0)),
            pl.BlockSpec((BLK, 1), lambda b: (b, 0)),
            pl.BlockSpec((BLK, 1), lambda b: (b, 0)),
        ],
        out_specs=pl.BlockSpec((BLK, H), lambda b: (b, 0)),
        out_shape=jax.ShapeDtypeStruct((T, H), jnp.float32),
    )(ys, g1, g2, w1c, w2c)


def kernel(x, shared_gate_w, shared_up_w, shared_down_w,
           routed_gate_w, routed_up_w, routed_down_w,
           router_w, routing_bias):
    x2d = x.reshape(T, H)
    bias2d = routing_bias.reshape(1, E)
    sgw = shared_gate_w.astype(jnp.bfloat16)
    suw = shared_up_w.astype(jnp.bfloat16)
    sdw = shared_down_w.astype(jnp.bfloat16)
    rgw = routed_gate_w.astype(jnp.bfloat16)
    ruw = routed_up_w.astype(jnp.bfloat16)
    rdw = routed_down_w.astype(jnp.bfloat16)

    e1c, e2c, w1c, w2c, cnt3 = _router(x2d, router_w, bias2d)
    e1 = e1c.reshape(T)
    e2 = e2c.reshape(T)
    cnt = cnt3.reshape(NW, L)

    disp, pos1, pos2, bex, nb = _dispatch(x2d, e1, e2, cnt)
    ys = _shared_ffn(x2d, sgw, suw, sdw)
    yr = _routed_ffn(bex, nb, disp, rgw, ruw, rdw)
    g1, g2 = _gather(yr, pos1, pos2)
    out2d = _combine(ys, g1, g2, w1c, w2c)
    return out2d.reshape(x.shape)


# speedup vs baseline: 1.2105x; 1.2105x over previous
import functools

import jax
import jax.numpy as jnp
from jax import lax
from jax.experimental import pallas as pl
from jax.experimental.pallas import tpu as pltpu
from jax.experimental.pallas import tpu_sc as plsc

H = 1024
I = 2048
E = 7
T = 4096
BLK = 256
NB = 39
NSLOTS = NB * BLK
NC, NS, L = 2, 16, 16
NW = NC * NS
TPW = T // NW


def _router_body(x_ref, rw_ref, bias_ref, e1_ref, e2_ref, w1_ref, w2_ref,
                 cnt_ref):
    xb = x_ref[...]
    logits = jnp.dot(xb, rw_ref[...], preferred_element_type=jnp.float32)
    probs = jax.nn.sigmoid(logits + bias_ref[...])
    lanes = lax.broadcasted_iota(jnp.int32, probs.shape, 1)
    m1 = jnp.max(probs, axis=1, keepdims=True)
    i1 = jnp.min(jnp.where(probs >= m1, lanes, 127), axis=1, keepdims=True)
    probs2 = jnp.where(lanes == i1, -1.0, probs)
    m2 = jnp.max(probs2, axis=1, keepdims=True)
    i2 = jnp.min(jnp.where(probs2 >= m2, lanes, 127), axis=1, keepdims=True)
    s = m1 + m2
    e1_ref[...] = i1
    e2_ref[...] = i2
    w1_ref[...] = m1 / s
    w2_ref[...] = m2 / s
    lanes16 = lax.broadcasted_iota(jnp.int32, (512, L), 1)
    onehot = ((lanes16 == i1) | (lanes16 == i2)).astype(jnp.float32)
    seg = lax.broadcasted_iota(jnp.int32, (4, 512), 0)
    tok = lax.broadcasted_iota(jnp.int32, (4, 512), 1)
    sel = (tok // TPW == seg).astype(jnp.float32)
    cnt = jnp.dot(sel, onehot, preferred_element_type=jnp.float32)
    cnt_ref[...] = cnt[None].astype(jnp.int32)


def _router(x2d, rw, bias2d):
    tb = 512
    grid = (T // tb,)
    return pl.pallas_call(
        _router_body,
        grid=grid,
        in_specs=[
            pl.BlockSpec((tb, H), lambda b: (b, 0)),
            pl.BlockSpec((H, E), lambda b: (0, 0)),
            pl.BlockSpec((1, E), lambda b: (0, 0)),
        ],
        out_specs=[
            pl.BlockSpec((tb, 1), lambda b: (b, 0)),
            pl.BlockSpec((tb, 1), lambda b: (b, 0)),
            pl.BlockSpec((tb, 1), lambda b: (b, 0)),
            pl.BlockSpec((tb, 1), lambda b: (b, 0)),
            pl.BlockSpec((1, 4, L), lambda b: (b, 0, 0)),
        ],
        out_shape=[
            jax.ShapeDtypeStruct((T, 1), jnp.int32),
            jax.ShapeDtypeStruct((T, 1), jnp.int32),
            jax.ShapeDtypeStruct((T, 1), jnp.float32),
            jax.ShapeDtypeStruct((T, 1), jnp.float32),
            jax.ShapeDtypeStruct((T // tb, 4, L), jnp.int32),
        ],
    )(x2d, rw, bias2d)


def _dispatch_body(x_hbm, e1_hbm, e2_hbm, cnt_hbm,
                   disp_hbm, pos1_hbm, pos2_hbm, bex_hbm, nb_hbm,
                   cnt_v, e1_v, e2_v, rows_v, rows2_v, bex_v, nb_v,
                   p1c0, p1c1, p1c2, p1c3, p2c0, p2c1, p2c2, p2c3,
                   sem, semla, semlb):
    cid = lax.axis_index("c")
    sid = lax.axis_index("s")
    wid = sid * NC + cid
    lane = lax.iota(jnp.int32, L)

    pltpu.sync_copy(cnt_hbm, cnt_v)
    totals = cnt_v[0, :]
    for w in range(1, NW):
        totals = totals + cnt_v[w, :]
    pad = ((totals + (BLK - 1)) >> 8) << 8
    cum_incl = plsc.cumsum(pad)
    start = cum_incl - pad
    base = start
    for w in range(NW):
        base = base + jnp.where(w < wid, cnt_v[w, :], 0)

    t0 = wid * TPW
    pltpu.sync_copy(e1_hbm.at[pl.ds(t0, TPW)], e1_v)
    pltpu.sync_copy(e2_hbm.at[pl.ds(t0, TPW)], e2_v)

    p1c = (p1c0, p1c1, p1c2, p1c3)
    p2c = (p2c0, p2c1, p2c2, p2c3)
    for stream, chunks in ((e1_v, p1c), (e2_v, p2c)):
        for v in range(TPW // L):
            ev = stream[pl.ds(v * L, L)]
            pos = jnp.zeros((L,), jnp.int32)
            for e in range(E):
                mask = ev == e
                incl = plsc.cumsum(jnp.where(mask, 1, 0))
                base_e = jnp.sum(jnp.where(lane == e, base, 0))
                pos = jnp.where(mask, base_e + incl - 1, pos)
                base = base + jnp.where(lane == e, jnp.max(incl), 0)
            chunks[v // 2][pl.ds((v % 2) * L, L)] = pos

    rows = (rows_v, rows2_v)
    semls = (semla, semlb)
    loads = {}
    scats = {}

    def fire_load(c):
        loads[c] = pltpu.async_copy(
            x_hbm.at[pl.ds(t0 + c * 32, 32)], rows[c % 2], semls[c % 2])

    fire_load(0)
    for c in range(4):
        if c < 4 - 1:
            if c - 1 >= 0:
                for dsc in scats.pop(c - 1):
                    dsc.wait()
            fire_load(c + 1)
        loads.pop(c).wait()
        scats[c] = (
            pltpu.async_copy(rows[c % 2], disp_hbm.at[p1c[c]], sem),
            pltpu.async_copy(rows[c % 2], disp_hbm.at[p2c[c]], sem),
        )
        tc = t0 + c * 32
        pltpu.sync_copy(p1c[c], pos1_hbm.at[pl.ds(tc, 32)])
        pltpu.sync_copy(p2c[c], pos2_hbm.at[pl.ds(tc, 32)])
    for c in (2, 3):
        for dsc in scats.pop(c):
            dsc.wait()

    @pl.when(wid == 0)
    def _():
        total_pad = jnp.max(jnp.where(lane < E, cum_incl, 0))
        for j in range(3):
            bidx = (lane + j * L) * BLK
            acc = jnp.zeros((L,), jnp.int32)
            for e in range(E):
                ci = jnp.sum(jnp.where(lane == e, cum_incl, 0))
                acc = acc + jnp.where(bidx >= ci, 1, 0)
            bex_v[pl.ds(j * L, L)] = jnp.minimum(acc, E - 1)
        nb_v[...] = jnp.where(lane == 0, total_pad >> 8, 0)
        pltpu.sync_copy(bex_v, bex_hbm)
        pltpu.sync_copy(nb_v, nb_hbm)


def _dispatch(x2d, e1, e2, cnt):
    mesh = plsc.VectorSubcoreMesh(core_axis_name="c", subcore_axis_name="s")
    f = pl.kernel(
        _dispatch_body,
        out_type=[
            jax.ShapeDtypeStruct((NSLOTS, H), jnp.float32),
            jax.ShapeDtypeStruct((T,), jnp.int32),
            jax.ShapeDtypeStruct((T,), jnp.int32),
            jax.ShapeDtypeStruct((48,), jnp.int32),
            jax.ShapeDtypeStruct((L,), jnp.int32),
        ],
        mesh=mesh,
        scratch_types=[
            pltpu.VMEM((NW, L), jnp.int32),
            pltpu.VMEM((TPW,), jnp.int32),
            pltpu.VMEM((TPW,), jnp.int32),
            pltpu.VMEM((32, H), jnp.float32),
            pltpu.VMEM((32, H), jnp.float32),
            pltpu.VMEM((48,), jnp.int32),
            pltpu.VMEM((L,), jnp.int32),
        ] + [pltpu.VMEM((32,), jnp.int32) for _ in range(8)]
        + [pltpu.SemaphoreType.DMA, pltpu.SemaphoreType.DMA,
           pltpu.SemaphoreType.DMA],
        compiler_params=pltpu.CompilerParams(needs_layout_passes=False),
    )
    return f(x2d, e1, e2, cnt)


def _ffn_body(x_ref, g_ref, u_ref, d_ref, y_ref):
    xb = x_ref[...]
    a = jnp.dot(xb, g_ref[...], preferred_element_type=jnp.float32)
    b = jnp.dot(xb, u_ref[...], preferred_element_type=jnp.float32)
    inter = jax.nn.silu(a) * b
    y_ref[...] = jnp.dot(inter, d_ref[...], preferred_element_type=jnp.float32)


def _shared_ffn(x2d, gw, uw, dw):
    grid = (T // BLK,)
    return pl.pallas_call(
        _ffn_body,
        grid=grid,
        in_specs=[
            pl.BlockSpec((BLK, H), lambda b: (b, 0)),
            pl.BlockSpec((H, I), lambda b: (0, 0)),
            pl.BlockSpec((H, I), lambda b: (0, 0)),
            pl.BlockSpec((I, H), lambda b: (0, 0)),
        ],
        out_specs=pl.BlockSpec((BLK, H), lambda b: (b, 0)),
        out_shape=jax.ShapeDtypeStruct((T, H), jnp.float32),
    )(x2d, gw, uw, dw)


def _routed_body(bex_ref, nb_ref, x_ref, g_ref, u_ref, d_ref, y_ref):
    b = pl.program_id(0)

    @pl.when(b < nb_ref[0])
    def _():
        xb = x_ref[...]
        a = jnp.dot(xb, g_ref[0], preferred_element_type=jnp.float32)
        u = jnp.dot(xb, u_ref[0], preferred_element_type=jnp.float32)
        inter = jax.nn.silu(a) * u
        y_ref[...] = jnp.dot(inter, d_ref[0],
                             preferred_element_type=jnp.float32)


def _routed_ffn(bex, nb, disp, gw, uw, dw):
    grid_spec = pltpu.PrefetchScalarGridSpec(
        num_scalar_prefetch=2,
        grid=(NB,),
        in_specs=[
            pl.BlockSpec((BLK, H), lambda b, bex, nb: (b, 0)),
            pl.BlockSpec((1, H, I), lambda b, bex, nb: (bex[b], 0, 0)),
            pl.BlockSpec((1, H, I), lambda b, bex, nb: (bex[b], 0, 0)),
            pl.BlockSpec((1, I, H), lambda b, bex, nb: (bex[b], 0, 0)),
        ],
        out_specs=pl.BlockSpec((BLK, H), lambda b, bex, nb: (b, 0)),
    )
    return pl.pallas_call(
        _routed_body,
        grid_spec=grid_spec,
        out_shape=jax.ShapeDtypeStruct((NSLOTS, H), jnp.float32),
    )(bex, nb, disp, gw, uw, dw)


def _gather_body(yr_hbm, pos1_hbm, pos2_hbm, g1_hbm, g2_hbm,
                 r1a_v, r1b_v, r2a_v, r2b_v, p1_v, p2_v,
                 p1sa, p1sb, p2sa, p2sb, sema, semb):
    cid = lax.axis_index("c")
    sid = lax.axis_index("s")
    wid = sid * NC + cid
    t0 = wid * TPW

    pltpu.sync_copy(pos1_hbm.at[pl.ds(t0, TPW)], p1_v)
    pltpu.sync_copy(pos2_hbm.at[pl.ds(t0, TPW)], p2_v)

    r1 = (r1a_v, r1b_v)
    r2 = (r2a_v, r2b_v)
    p1s = (p1sa, p1sb)
    p2s = (p2sa, p2sb)
    sems = (sema, semb)
    NCH = TPW // L
    pending = {}

    def fire(c):
        b = c % 2
        p1s[b][...] = p1_v[pl.ds(c * L, L)]
        p2s[b][...] = p2_v[pl.ds(c * L, L)]
        pending[c] = (
            pltpu.async_copy(yr_hbm.at[p1s[b]], r1[b], sems[b]),
            pltpu.async_copy(yr_hbm.at[p2s[b]], r2[b], sems[b]),
        )

    fire(0)
    for c in range(NCH):
        b = c % 2
        if c < NCH - 1:
            fire(c + 1)
        for dsc in pending.pop(c):
            dsc.wait()
        sl = pl.ds(t0 + c * L, L)
        pltpu.sync_copy(r1[b], g1_hbm.at[sl])
        pltpu.sync_copy(r2[b], g2_hbm.at[sl])


def _gather(yr, pos1, pos2):
    mesh = plsc.VectorSubcoreMesh(core_axis_name="c", subcore_axis_name="s")
    f = pl.kernel(
        _gather_body,
        out_type=[
            jax.ShapeDtypeStruct((T, H), jnp.float32),
            jax.ShapeDtypeStruct((T, H), jnp.float32),
        ],
        mesh=mesh,
        scratch_types=[
            pltpu.VMEM((L, H), jnp.float32),
            pltpu.VMEM((L, H), jnp.float32),
            pltpu.VMEM((L, H), jnp.float32),
            pltpu.VMEM((L, H), jnp.float32),
            pltpu.VMEM((TPW,), jnp.int32),
            pltpu.VMEM((TPW,), jnp.int32),
            pltpu.VMEM((L,), jnp.int32),
            pltpu.VMEM((L,), jnp.int32),
            pltpu.VMEM((L,), jnp.int32),
            pltpu.VMEM((L,), jnp.int32),
            pltpu.SemaphoreType.DMA,
            pltpu.SemaphoreType.DMA,
        ],
        compiler_params=pltpu.CompilerParams(needs_layout_passes=False),
    )
    return f(yr, pos1, pos2)


def _combine_body(ys_ref, g1_ref, g2_ref, w1_ref, w2_ref, out_ref):
    out_ref[...] = (ys_ref[...] + w1_ref[...] * g1_ref[...]
                    + w2_ref[...] * g2_ref[...])


def _combine(ys, g1, g2, w1c, w2c):
    grid = (T // BLK,)
    return pl.pallas_call(
        _combine_body,
        grid=grid,
        in_specs=[
            pl.BlockSpec((BLK, H), lambda b: (b, 0)),
            pl.BlockSpec((BLK, H), lambda b: (b, 0)),
            pl.BlockSpec((BLK, H), lambda b: (b, 0)),
            pl.BlockSpec((BLK, 1), lambda b: (b, 0)),
            pl.BlockSpec((BLK, 1), lambda b: (b, 0)),
        ],
        out_specs=pl.BlockSpec((BLK, H), lambda b: (b, 0)),
        out_shape=jax.ShapeDtypeStruct((T, H), jnp.float32),
    )(ys, g1, g2, w1c, w2c)


def kernel(x, shared_gate_w, shared_up_w, shared_down_w,
           routed_gate_w, routed_up_w, routed_down_w,
           router_w, routing_bias):
    x2d = x.reshape(T, H)
    bias2d = routing_bias.reshape(1, E)

    e1c, e2c, w1c, w2c, cnt3 = _router(x2d, router_w, bias2d)
    e1 = e1c.reshape(T)
    e2 = e2c.reshape(T)
    cnt = cnt3.reshape(NW, L)

    disp, pos1, pos2, bex, nb = _dispatch(x2d, e1, e2, cnt)
    ys = _shared_ffn(x2d, shared_gate_w, shared_up_w, shared_down_w)
    yr = _routed_ffn(bex, nb, disp, routed_gate_w, routed_up_w,
                     routed_down_w)
    g1, g2 = _gather(yr, pos1, pos2)
    out2d = _combine(ys, g1, g2, w1c, w2c)
    return out2d.reshape(x.shape)


# speedup vs baseline: 1.2111x; 1.0005x over previous
import functools

import jax
import jax.numpy as jnp
from jax import lax
from jax.experimental import pallas as pl
from jax.experimental.pallas import tpu as pltpu
from jax.experimental.pallas import tpu_sc as plsc

H = 1024
I = 2048
E = 7
T = 4096
BLK = 256
NB = 39
NSLOTS = NB * BLK
NC, NS, L = 2, 16, 16
NW = NC * NS
TPW = T // NW


def _router_body(x_ref, rw_ref, bias_ref, e1_ref, e2_ref, w1_ref, w2_ref,
                 cnt_ref):
    xb = x_ref[...]
    logits = jnp.dot(xb, rw_ref[...], preferred_element_type=jnp.float32)
    probs = jax.nn.sigmoid(logits + bias_ref[...])
    lanes = lax.broadcasted_iota(jnp.int32, probs.shape, 1)
    m1 = jnp.max(probs, axis=1, keepdims=True)
    i1 = jnp.min(jnp.where(probs >= m1, lanes, 127), axis=1, keepdims=True)
    probs2 = jnp.where(lanes == i1, -1.0, probs)
    m2 = jnp.max(probs2, axis=1, keepdims=True)
    i2 = jnp.min(jnp.where(probs2 >= m2, lanes, 127), axis=1, keepdims=True)
    s = m1 + m2
    e1_ref[...] = i1
    e2_ref[...] = i2
    w1_ref[...] = m1 / s
    w2_ref[...] = m2 / s
    lanes16 = lax.broadcasted_iota(jnp.int32, (512, L), 1)
    onehot = ((lanes16 == i1) | (lanes16 == i2)).astype(jnp.float32)
    seg = lax.broadcasted_iota(jnp.int32, (4, 512), 0)
    tok = lax.broadcasted_iota(jnp.int32, (4, 512), 1)
    sel = (tok // TPW == seg).astype(jnp.float32)
    cnt = jnp.dot(sel, onehot, preferred_element_type=jnp.float32)
    cnt_ref[...] = cnt[None].astype(jnp.int32)


def _router(x2d, rw, bias2d):
    tb = 512
    grid = (T // tb,)
    return pl.pallas_call(
        _router_body,
        grid=grid,
        in_specs=[
            pl.BlockSpec((tb, H), lambda b: (b, 0)),
            pl.BlockSpec((H, E), lambda b: (0, 0)),
            pl.BlockSpec((1, E), lambda b: (0, 0)),
        ],
        out_specs=[
            pl.BlockSpec((tb, 1), lambda b: (b, 0)),
            pl.BlockSpec((tb, 1), lambda b: (b, 0)),
            pl.BlockSpec((tb, 1), lambda b: (b, 0)),
            pl.BlockSpec((tb, 1), lambda b: (b, 0)),
            pl.BlockSpec((1, 4, L), lambda b: (b, 0, 0)),
        ],
        out_shape=[
            jax.ShapeDtypeStruct((T, 1), jnp.int32),
            jax.ShapeDtypeStruct((T, 1), jnp.int32),
            jax.ShapeDtypeStruct((T, 1), jnp.float32),
            jax.ShapeDtypeStruct((T, 1), jnp.float32),
            jax.ShapeDtypeStruct((T // tb, 4, L), jnp.int32),
        ],
    )(x2d, rw, bias2d)


def _dispatch_body(x_hbm, e1_hbm, e2_hbm, cnt_hbm,
                   disp_hbm, pos1_hbm, pos2_hbm, bex_hbm, nb_hbm,
                   cnt_v, e1_v, e2_v, rows_v, rows2_v, bex_v, nb_v,
                   p1c0, p1c1, p1c2, p1c3, p2c0, p2c1, p2c2, p2c3,
                   sem, semla, semlb):
    cid = lax.axis_index("c")
    sid = lax.axis_index("s")
    wid = sid * NC + cid
    lane = lax.iota(jnp.int32, L)

    pltpu.sync_copy(cnt_hbm, cnt_v)
    totals = cnt_v[0, :]
    for w in range(1, NW):
        totals = totals + cnt_v[w, :]
    pad = ((totals + (BLK - 1)) >> 8) << 8
    cum_incl = plsc.cumsum(pad)
    start = cum_incl - pad
    base = start
    for w in range(NW):
        base = base + jnp.where(w < wid, cnt_v[w, :], 0)

    t0 = wid * TPW
    pltpu.sync_copy(e1_hbm.at[pl.ds(t0, TPW)], e1_v)
    pltpu.sync_copy(e2_hbm.at[pl.ds(t0, TPW)], e2_v)

    p1c = (p1c0, p1c1, p1c2, p1c3)
    p2c = (p2c0, p2c1, p2c2, p2c3)
    for stream, chunks in ((e1_v, p1c), (e2_v, p2c)):
        for v in range(TPW // L):
            ev = stream[pl.ds(v * L, L)]
            pos = jnp.zeros((L,), jnp.int32)
            for e in range(E):
                mask = ev == e
                incl = plsc.cumsum(jnp.where(mask, 1, 0))
                base_e = jnp.sum(jnp.where(lane == e, base, 0))
                pos = jnp.where(mask, base_e + incl - 1, pos)
                base = base + jnp.where(lane == e, jnp.max(incl), 0)
            chunks[v // 2][pl.ds((v % 2) * L, L)] = pos

    rows = (rows_v, rows2_v)
    semls = (semla, semlb)
    loads = {}
    scats = {}

    def fire_load(c):
        loads[c] = pltpu.async_copy(
            x_hbm.at[pl.ds(t0 + c * 32, 32)], rows[c % 2], semls[c % 2])

    fire_load(0)
    for c in range(4):
        if c < 4 - 1:
            if c - 1 >= 0:
                for dsc in scats.pop(c - 1):
                    dsc.wait()
            fire_load(c + 1)
        loads.pop(c).wait()
        scats[c] = (
            pltpu.async_copy(rows[c % 2], disp_hbm.at[p1c[c]], sem),
            pltpu.async_copy(rows[c % 2], disp_hbm.at[p2c[c]], sem),
        )
        tc = t0 + c * 32
        pltpu.sync_copy(p1c[c], pos1_hbm.at[pl.ds(tc, 32)])
        pltpu.sync_copy(p2c[c], pos2_hbm.at[pl.ds(tc, 32)])
    for c in (2, 3):
        for dsc in scats.pop(c):
            dsc.wait()

    @pl.when(wid == 0)
    def _():
        total_pad = jnp.max(jnp.where(lane < E, cum_incl, 0))
        for j in range(3):
            bidx = (lane + j * L) * BLK
            acc = jnp.zeros((L,), jnp.int32)
            for e in range(E):
                ci = jnp.sum(jnp.where(lane == e, cum_incl, 0))
                acc = acc + jnp.where(bidx >= ci, 1, 0)
            bex_v[pl.ds(j * L, L)] = jnp.minimum(acc, E - 1)
        nb_v[...] = jnp.where(lane == 0, total_pad >> 8, 0)
        pltpu.sync_copy(bex_v, bex_hbm)
        pltpu.sync_copy(nb_v, nb_hbm)


def _dispatch(x2d, e1, e2, cnt):
    mesh = plsc.VectorSubcoreMesh(core_axis_name="c", subcore_axis_name="s")
    f = pl.kernel(
        _dispatch_body,
        out_type=[
            jax.ShapeDtypeStruct((NSLOTS, H), jnp.float32),
            jax.ShapeDtypeStruct((T,), jnp.int32),
            jax.ShapeDtypeStruct((T,), jnp.int32),
            jax.ShapeDtypeStruct((48,), jnp.int32),
            jax.ShapeDtypeStruct((L,), jnp.int32),
        ],
        mesh=mesh,
        scratch_types=[
            pltpu.VMEM((NW, L), jnp.int32),
            pltpu.VMEM((TPW,), jnp.int32),
            pltpu.VMEM((TPW,), jnp.int32),
            pltpu.VMEM((32, H), jnp.float32),
            pltpu.VMEM((32, H), jnp.float32),
            pltpu.VMEM((48,), jnp.int32),
            pltpu.VMEM((L,), jnp.int32),
        ] + [pltpu.VMEM((32,), jnp.int32) for _ in range(8)]
        + [pltpu.SemaphoreType.DMA, pltpu.SemaphoreType.DMA,
           pltpu.SemaphoreType.DMA],
        compiler_params=pltpu.CompilerParams(needs_layout_passes=False),
    )
    return f(x2d, e1, e2, cnt)


def _ffn_body(x_ref, g_ref, u_ref, d_ref, y_ref):
    xb = x_ref[...]
    a = jnp.dot(xb, g_ref[...], preferred_element_type=jnp.float32)
    b = jnp.dot(xb, u_ref[...], preferred_element_type=jnp.float32)
    inter = jax.nn.silu(a) * b
    y_ref[...] = jnp.dot(inter, d_ref[...], preferred_element_type=jnp.float32)


def _shared_ffn(x2d, gw, uw, dw):
    grid = (T // BLK,)
    return pl.pallas_call(
        _ffn_body,
        grid=grid,
        in_specs=[
            pl.BlockSpec((BLK, H), lambda b: (b, 0)),
            pl.BlockSpec((H, I), lambda b: (0, 0)),
            pl.BlockSpec((H, I), lambda b: (0, 0)),
            pl.BlockSpec((I, H), lambda b: (0, 0)),
        ],
        out_specs=pl.BlockSpec((BLK, H), lambda b: (b, 0)),
        out_shape=jax.ShapeDtypeStruct((T, H), jnp.float32),
        compiler_params=pltpu.CompilerParams(
            vmem_limit_bytes=128 * 1024 * 1024),
    )(x2d, gw, uw, dw)


def _routed_body(bex_ref, nb_ref, x_ref, g_ref, u_ref, d_ref, y_ref):
    b = pl.program_id(0)

    @pl.when(b < nb_ref[0])
    def _():
        xb = x_ref[...]
        a = jnp.dot(xb, g_ref[0], preferred_element_type=jnp.float32)
        u = jnp.dot(xb, u_ref[0], preferred_element_type=jnp.float32)
        inter = jax.nn.silu(a) * u
        y_ref[...] = jnp.dot(inter, d_ref[0],
                             preferred_element_type=jnp.float32)


def _routed_ffn(bex, nb, disp, gw, uw, dw):
    grid_spec = pltpu.PrefetchScalarGridSpec(
        num_scalar_prefetch=2,
        grid=(NB,),
        in_specs=[
            pl.BlockSpec((BLK, H), lambda b, bex, nb: (b, 0)),
            pl.BlockSpec((1, H, I), lambda b, bex, nb: (bex[b], 0, 0)),
            pl.BlockSpec((1, H, I), lambda b, bex, nb: (bex[b], 0, 0)),
            pl.BlockSpec((1, I, H), lambda b, bex, nb: (bex[b], 0, 0)),
        ],
        out_specs=pl.BlockSpec((BLK, H), lambda b, bex, nb: (b, 0)),
    )
    return pl.pallas_call(
        _routed_body,
        grid_spec=grid_spec,
        out_shape=jax.ShapeDtypeStruct((NSLOTS, H), jnp.float32),
        compiler_params=pltpu.CompilerParams(
            vmem_limit_bytes=128 * 1024 * 1024),
    )(bex, nb, disp, gw, uw, dw)


def _gather_body(yr_hbm, pos1_hbm, pos2_hbm, g1_hbm, g2_hbm,
                 r1a_v, r1b_v, r2a_v, r2b_v, p1_v, p2_v,
                 p1sa, p1sb, p2sa, p2sb, sema, semb):
    cid = lax.axis_index("c")
    sid = lax.axis_index("s")
    wid = sid * NC + cid
    t0 = wid * TPW

    pltpu.sync_copy(pos1_hbm.at[pl.ds(t0, TPW)], p1_v)
    pltpu.sync_copy(pos2_hbm.at[pl.ds(t0, TPW)], p2_v)

    r1 = (r1a_v, r1b_v)
    r2 = (r2a_v, r2b_v)
    p1s = (p1sa, p1sb)
    p2s = (p2sa, p2sb)
    sems = (sema, semb)
    NCH = TPW // L
    pending = {}

    def fire(c):
        b = c % 2
        p1s[b][...] = p1_v[pl.ds(c * L, L)]
        p2s[b][...] = p2_v[pl.ds(c * L, L)]
        pending[c] = (
            pltpu.async_copy(yr_hbm.at[p1s[b]], r1[b], sems[b]),
            pltpu.async_copy(yr_hbm.at[p2s[b]], r2[b], sems[b]),
        )

    fire(0)
    for c in range(NCH):
        b = c % 2
        if c < NCH - 1:
            fire(c + 1)
        for dsc in pending.pop(c):
            dsc.wait()
        sl = pl.ds(t0 + c * L, L)
        pltpu.sync_copy(r1[b], g1_hbm.at[sl])
        pltpu.sync_copy(r2[b], g2_hbm.at[sl])


def _gather(yr, pos1, pos2):
    mesh = plsc.VectorSubcoreMesh(core_axis_name="c", subcore_axis_name="s")
    f = pl.kernel(
        _gather_body,
        out_type=[
            jax.ShapeDtypeStruct((T, H), jnp.float32),
            jax.ShapeDtypeStruct((T, H), jnp.float32),
        ],
        mesh=mesh,
        scratch_types=[
            pltpu.VMEM((L, H), jnp.float32),
            pltpu.VMEM((L, H), jnp.float32),
            pltpu.VMEM((L, H), jnp.float32),
            pltpu.VMEM((L, H), jnp.float32),
            pltpu.VMEM((TPW,), jnp.int32),
            pltpu.VMEM((TPW,), jnp.int32),
            pltpu.VMEM((L,), jnp.int32),
            pltpu.VMEM((L,), jnp.int32),
            pltpu.VMEM((L,), jnp.int32),
            pltpu.VMEM((L,), jnp.int32),
            pltpu.SemaphoreType.DMA,
            pltpu.SemaphoreType.DMA,
        ],
        compiler_params=pltpu.CompilerParams(needs_layout_passes=False),
    )
    return f(yr, pos1, pos2)


def _combine_body(ys_ref, g1_ref, g2_ref, w1_ref, w2_ref, out_ref):
    out_ref[...] = (ys_ref[...] + w1_ref[...] * g1_ref[...]
                    + w2_ref[...] * g2_ref[...])


def _combine(ys, g1, g2, w1c, w2c):
    grid = (T // BLK,)
    return pl.pallas_call(
        _combine_body,
        grid=grid,
        in_specs=[
            pl.BlockSpec((BLK, H), lambda b: (b, 0)),
            pl.BlockSpec((BLK, H), lambda b: (b, 0)),
            pl.BlockSpec((BLK, H), lambda b: (b, 0)),
            pl.BlockSpec((BLK, 1), lambda b: (b, 0)),
            pl.BlockSpec((BLK, 1), lambda b: (b, 0)),
        ],
        out_specs=pl.BlockSpec((BLK, H), lambda b: (b, 0)),
        out_shape=jax.ShapeDtypeStruct((T, H), jnp.float32),
    )(ys, g1, g2, w1c, w2c)


def kernel(x, shared_gate_w, shared_up_w, shared_down_w,
           routed_gate_w, routed_up_w, routed_down_w,
           router_w, routing_bias):
    x2d = x.reshape(T, H)
    bias2d = routing_bias.reshape(1, E)

    e1c, e2c, w1c, w2c, cnt3 = _router(x2d, router_w, bias2d)
    e1 = e1c.reshape(T)
    e2 = e2c.reshape(T)
    cnt = cnt3.reshape(NW, L)

    disp, pos1, pos2, bex, nb = _dispatch(x2d, e1, e2, cnt)
    ys = _shared_ffn(x2d, shared_gate_w, shared_up_w, shared_down_w)
    yr = _routed_ffn(bex, nb, disp, routed_gate_w, routed_up_w,
                     routed_down_w)
    g1, g2 = _gather(yr, pos1, pos2)
    out2d = _combine(ys, g1, g2, w1c, w2c)
    return out2d.reshape(x.shape)
